# baseline (speedup 1.0000x reference)
import functools
import jax
import jax.numpy as jnp
from jax import lax
from jax.experimental import pallas as pl
from jax.experimental.pallas import tpu as pltpu
from jax.experimental.pallas import tpu_sc as plsc

N_NODES = 10000
N_EDGES = 320000
D = 128

NC = 2
NS = 16
NW = NC * NS
L = 16

EPW = N_EDGES // NW
K2_CH = 80
NPW = 320
NPAD = NW * NPW
K4_CE = 4000
UNR = 2
IDN = 128
GB = IDN - UNR * L
TRASH = NPW

_mesh = plsc.VectorSubcoreMesh(core_axis_name="c", subcore_axis_name="s")
_sc_params = pltpu.CompilerParams(needs_layout_passes=False)


def _k1_body(x_ref, wd_ref, wb_ref, b1_ref, a_ref, b_ref):
    xa = x_ref[...]
    a_ref[...] = (jnp.dot(xa, wd_ref[...], preferred_element_type=jnp.float32)
                  + b1_ref[...])
    b_ref[...] = jnp.dot(xa, wb_ref[...], preferred_element_type=jnp.float32)


def _k1(x, Wd, Wb, b1row):
    grid = 10
    blk = N_NODES // grid
    return pl.pallas_call(
        _k1_body,
        grid=(grid,),
        in_specs=[
            pl.BlockSpec((blk, D), lambda i: (i, 0)),
            pl.BlockSpec((D, D), lambda i: (0, 0)),
            pl.BlockSpec((D, D), lambda i: (0, 0)),
            pl.BlockSpec((1, D), lambda i: (0, 0)),
        ],
        out_specs=[
            pl.BlockSpec((blk, D), lambda i: (i, 0)),
            pl.BlockSpec((blk, D), lambda i: (i, 0)),
        ],
        out_shape=[
            jax.ShapeDtypeStruct((N_NODES, D), jnp.float32),
            jax.ShapeDtypeStruct((N_NODES, D), jnp.float32),
        ],
    )(x, Wd, Wb, b1row)


@functools.partial(
    pl.kernel,
    out_type=jax.ShapeDtypeStruct((N_EDGES, D), jnp.float32),
    mesh=_mesh,
    compiler_params=_sc_params,
    scratch_types=[
        pltpu.VMEM((EPW,), jnp.int32),
        pltpu.VMEM((EPW,), jnp.int32),
        pltpu.VMEM((K2_CH, D), jnp.float32),
        pltpu.VMEM((K2_CH, D), jnp.float32),
        pltpu.VMEM((K2_CH, D), jnp.float32),
        pltpu.VMEM((K2_CH, D), jnp.float32),
        pltpu.SemaphoreType.DMA,
        pltpu.SemaphoreType.DMA,
        pltpu.SemaphoreType.DMA,
        pltpu.SemaphoreType.DMA,
    ],
)
def _k2(a_hbm, b_hbm, dst_hbm, src_hbm, out_hbm, dsti, srci,
        ga0, gb0, ga1, gb1, sa0, sb0, sa1, sb1):
    wid = lax.axis_index("s") * NC + lax.axis_index("c")
    base = wid * EPW
    pltpu.sync_copy(dst_hbm.at[pl.ds(base, EPW)], dsti)
    pltpu.sync_copy(src_hbm.at[pl.ds(base, EPW)], srci)

    def add_and_store(ga, gb, ci):
        def addrow(r, c2):
            for j in range(D // L):
                sl = pl.ds(j * L, L)
                ga[r, sl] = ga[r, sl] + gb[r, sl]
            return c2

        lax.fori_loop(0, K2_CH, addrow, 0)
        pltpu.sync_copy(ga, out_hbm.at[pl.ds(base + ci * K2_CH, K2_CH)])

    def pair(c2, carry):
        ci0 = c2 * 2
        ci1 = ci0 + 1
        ca0 = pltpu.async_copy(
            a_hbm.at[dsti.at[pl.ds(ci0 * K2_CH, K2_CH)]], ga0, sa0)
        cb0 = pltpu.async_copy(
            b_hbm.at[srci.at[pl.ds(ci0 * K2_CH, K2_CH)]], gb0, sb0)
        ca1 = pltpu.async_copy(
            a_hbm.at[dsti.at[pl.ds(ci1 * K2_CH, K2_CH)]], ga1, sa1)
        cb1 = pltpu.async_copy(
            b_hbm.at[srci.at[pl.ds(ci1 * K2_CH, K2_CH)]], gb1, sb1)
        ca0.wait()
        cb0.wait()
        add_and_store(ga0, gb0, ci0)
        ca1.wait()
        cb1.wait()
        add_and_store(ga1, gb1, ci1)
        return carry

    NCH = EPW // K2_CH
    lax.fori_loop(0, NCH // 2, pair, 0)
    ci_t = NCH - 1
    ca = pltpu.async_copy(
        a_hbm.at[dsti.at[pl.ds(ci_t * K2_CH, K2_CH)]], ga0, sa0)
    cb = pltpu.async_copy(
        b_hbm.at[srci.at[pl.ds(ci_t * K2_CH, K2_CH)]], gb0, sb0)
    ca.wait()
    cb.wait()
    add_and_store(ga0, gb0, ci_t)


def _k3_body(h_ref, w2t_ref, b2_ref, o_ref):
    h = h_ref[...]
    h = h * jnp.tanh(jax.nn.softplus(h))
    o_ref[...] = (jnp.dot(h, w2t_ref[...], preferred_element_type=jnp.float32)
                  + b2_ref[...])


def _k3(h1, W2t, b2row):
    grid = 100
    blk = N_EDGES // grid
    return pl.pallas_call(
        _k3_body,
        grid=(grid,),
        in_specs=[
            pl.BlockSpec((blk, D), lambda i: (i, 0)),
            pl.BlockSpec((D, D), lambda i: (0, 0)),
            pl.BlockSpec((1, D), lambda i: (0, 0)),
        ],
        out_specs=pl.BlockSpec((blk, D), lambda i: (i, 0)),
        out_shape=jax.ShapeDtypeStruct((N_EDGES, D), jnp.float32),
    )(h1, W2t, b2row)


@functools.partial(
    pl.kernel,
    out_type=jax.ShapeDtypeStruct((NPAD, D), jnp.float32),
    mesh=_mesh,
    compiler_params=_sc_params,
    scratch_types=[
        pltpu.VMEM((K4_CE,), jnp.int32),
        pltpu.VMEM((IDN,), jnp.int32),
        pltpu.VMEM((IDN,), jnp.int32),
        pltpu.VMEM((IDN, D), jnp.float32),
        pltpu.VMEM((NPW + 8, D), jnp.float32),
        pltpu.SemaphoreType.DMA,
    ],
)
def _k4(h2_hbm, dst_hbm, out_hbm, dstbuf, idbuf, ldbuf, rows, acc, sem):
    wid = lax.axis_index("s") * NC + lax.axis_index("c")
    lo = wid * NPW
    neg = jnp.full((L,), -jnp.inf, dtype=jnp.float32)
    trash_v = jnp.full((L,), TRASH, dtype=jnp.int32)
    iota = lax.iota(jnp.int32, L)
    ones = jnp.full((L,), 1, jnp.int32)
    zeros = jnp.zeros((L,), jnp.int32)
    npw_v = jnp.full((L,), NPW, jnp.int32)
    lov = jnp.full((L,), lo, jnp.int32)

    def initacc(r, c):
        for j in range(D // L):
            acc[r, pl.ds(j * L, L)] = neg
        return c

    lax.fori_loop(0, NPW + 8, initacc, 0)
    for k in range(IDN // L):
        idbuf[pl.ds(k * L, L)] = zeros
        ldbuf[pl.ds(k * L, L)] = trash_v

    def flush():
        pltpu.async_copy(h2_hbm.at[idbuf], rows, sem).wait()

        def maxgroup(g, c):
            base = g * L
            ldv = ldbuf[pl.ds(base, L)]
            for lane in range(L):
                ld = ldv[lane]
                for j in range(D // L):
                    sl = pl.ds(j * L, L)
                    acc[ld, sl] = jnp.maximum(acc[ld, sl],
                                              rows[base + lane, sl])
            return c

        lax.fori_loop(0, IDN // L, maxgroup, 0)

    def chunk(ci, off):
        pltpu.sync_copy(dst_hbm.at[pl.ds(ci * K4_CE, K4_CE)], dstbuf)

        def step(i, off2):
            offu = off2
            for u in range(UNR):
                vbase = (i * UNR + u) * L
                dv = dstbuf[pl.ds(vbase, L)]
                ld = dv - lov
                m = (ld >= zeros) & (ld < npw_v)
                mi = jnp.where(m, ones, zeros)
                cs = plsc.cumsum(mi)
                pos = cs + jnp.full((L,), offu - 1, jnp.int32)
                eid = jnp.full((L,), ci * K4_CE + vbase, jnp.int32) + iota
                plsc.store_scatter(idbuf, [pos], eid, mask=m)
                plsc.store_scatter(ldbuf, [pos], ld, mask=m)
                offu = offu + cs[L - 1]
            do_flush = offu >= GB

            @pl.when(do_flush)
            def _():
                flush()

            return jnp.where(do_flush, 0, offu)

        return lax.fori_loop(0, K4_CE // (UNR * L), step, off)

    off = lax.fori_loop(0, N_EDGES // K4_CE, chunk, jnp.int32(0))

    @pl.when(off > 0)
    def _():
        flush()

    pltpu.sync_copy(acc.at[pl.ds(0, NPW)], out_hbm.at[pl.ds(lo, NPW)])


def _k5_body(agg_ref, g_ref, b_ref, o_ref):
    a = agg_ref[...]
    rows = lax.broadcasted_iota(jnp.int32, (NPAD, 1), 0)
    valid = rows < N_NODES
    a = jnp.where(jnp.isfinite(a), a, 0.0)
    a = jnp.where(valid, a, 0.0)
    inv_n = 1.0 / N_NODES
    mean = jnp.sum(a, axis=0, keepdims=True) * inv_n
    d = jnp.where(valid, a - mean, 0.0)
    var = jnp.sum(d * d, axis=0, keepdims=True) * inv_n
    scale = g_ref[...] * lax.rsqrt(var + 1e-5)
    o_ref[...] = (a[0:N_NODES] - mean) * scale + b_ref[...]


def _k5(aggp, grow, brow):
    return pl.pallas_call(
        _k5_body,
        out_shape=jax.ShapeDtypeStruct((N_NODES, D), jnp.float32),
    )(aggp, grow, brow)


@jax.jit
def kernel(x, edge_index, edge_attr, W1, b1, W2, b2, gamma, beta):
    W1a = W1[:, :D]
    W1b = W1[:, D:]
    Wd = (W1a - W1b).T
    Wb = W1b.T
    src = edge_index[0]
    dst = edge_index[1]
    A, B = _k1(x, Wd, Wb, b1[None, :])
    h1 = _k2(A, B, dst, src)
    h2 = _k3(h1, W2.T, b2[None, :])
    aggp = _k4(h2, dst)
    out = _k5(aggp, gamma[None, :], beta[None, :])
    return (out, edge_index, edge_attr)


# speedup vs baseline: 2.0004x; 2.0004x over previous
import functools
import jax
import jax.numpy as jnp
from jax import lax
from jax.experimental import pallas as pl
from jax.experimental.pallas import tpu as pltpu
from jax.experimental.pallas import tpu_sc as plsc

N_NODES = 10000
N_EDGES = 320000
D = 128

NC = 2
NS = 16
NW = NC * NS
L = 16

EPW = N_EDGES // NW
K2_CH = 80
NPW = 320
NPAD = NW * NPW
K4_CE = 4000
UNR = 1
IDN = 128
GB = IDN - UNR * L
TRASH = NPW

_mesh = plsc.VectorSubcoreMesh(core_axis_name="c", subcore_axis_name="s")
_sc_params = pltpu.CompilerParams(needs_layout_passes=False)


def _k1_body(x_ref, wd_ref, wb_ref, b1_ref, a_ref, b_ref):
    xa = x_ref[...]
    a_ref[...] = (jnp.dot(xa, wd_ref[...], preferred_element_type=jnp.float32)
                  + b1_ref[...])
    b_ref[...] = jnp.dot(xa, wb_ref[...], preferred_element_type=jnp.float32)


def _k1(x, Wd, Wb, b1row):
    grid = 10
    blk = N_NODES // grid
    return pl.pallas_call(
        _k1_body,
        grid=(grid,),
        in_specs=[
            pl.BlockSpec((blk, D), lambda i: (i, 0)),
            pl.BlockSpec((D, D), lambda i: (0, 0)),
            pl.BlockSpec((D, D), lambda i: (0, 0)),
            pl.BlockSpec((1, D), lambda i: (0, 0)),
        ],
        out_specs=[
            pl.BlockSpec((blk, D), lambda i: (i, 0)),
            pl.BlockSpec((blk, D), lambda i: (i, 0)),
        ],
        out_shape=[
            jax.ShapeDtypeStruct((N_NODES, D), jnp.float32),
            jax.ShapeDtypeStruct((N_NODES, D), jnp.float32),
        ],
    )(x, Wd, Wb, b1row)


@functools.partial(
    pl.kernel,
    out_type=jax.ShapeDtypeStruct((N_EDGES, D), jnp.float32),
    mesh=_mesh,
    compiler_params=_sc_params,
    scratch_types=[
        pltpu.VMEM((EPW,), jnp.int32),
        pltpu.VMEM((EPW,), jnp.int32),
        pltpu.VMEM((K2_CH, D), jnp.float32),
        pltpu.VMEM((K2_CH, D), jnp.float32),
        pltpu.VMEM((K2_CH, D), jnp.float32),
        pltpu.VMEM((K2_CH, D), jnp.float32),
        pltpu.SemaphoreType.DMA,
        pltpu.SemaphoreType.DMA,
        pltpu.SemaphoreType.DMA,
        pltpu.SemaphoreType.DMA,
    ],
)
def _k2(a_hbm, b_hbm, dst_hbm, src_hbm, out_hbm, dsti, srci,
        ga0, gb0, ga1, gb1, sa0, sb0, sa1, sb1):
    wid = lax.axis_index("s") * NC + lax.axis_index("c")
    base = wid * EPW
    pltpu.sync_copy(dst_hbm.at[pl.ds(base, EPW)], dsti)
    pltpu.sync_copy(src_hbm.at[pl.ds(base, EPW)], srci)

    def add_and_store(ga, gb, ci):
        def addrow(r, c2):
            for j in range(D // L):
                sl = pl.ds(j * L, L)
                ga[r, sl] = ga[r, sl] + gb[r, sl]
            return c2

        lax.fori_loop(0, K2_CH, addrow, 0)
        pltpu.sync_copy(ga, out_hbm.at[pl.ds(base + ci * K2_CH, K2_CH)])

    def pair(c2, carry):
        ci0 = c2 * 2
        ci1 = ci0 + 1
        ca0 = pltpu.async_copy(
            a_hbm.at[dsti.at[pl.ds(ci0 * K2_CH, K2_CH)]], ga0, sa0)
        cb0 = pltpu.async_copy(
            b_hbm.at[srci.at[pl.ds(ci0 * K2_CH, K2_CH)]], gb0, sb0)
        ca1 = pltpu.async_copy(
            a_hbm.at[dsti.at[pl.ds(ci1 * K2_CH, K2_CH)]], ga1, sa1)
        cb1 = pltpu.async_copy(
            b_hbm.at[srci.at[pl.ds(ci1 * K2_CH, K2_CH)]], gb1, sb1)
        ca0.wait()
        cb0.wait()
        add_and_store(ga0, gb0, ci0)
        ca1.wait()
        cb1.wait()
        add_and_store(ga1, gb1, ci1)
        return carry

    NCH = EPW // K2_CH
    lax.fori_loop(0, NCH // 2, pair, 0)
    ci_t = NCH - 1
    ca = pltpu.async_copy(
        a_hbm.at[dsti.at[pl.ds(ci_t * K2_CH, K2_CH)]], ga0, sa0)
    cb = pltpu.async_copy(
        b_hbm.at[srci.at[pl.ds(ci_t * K2_CH, K2_CH)]], gb0, sb0)
    ca.wait()
    cb.wait()
    add_and_store(ga0, gb0, ci_t)


def _k3_body(h_ref, w2t_ref, b2_ref, o_ref):
    h = h_ref[...]
    h = h * jnp.tanh(jax.nn.softplus(h))
    o_ref[...] = (jnp.dot(h, w2t_ref[...], preferred_element_type=jnp.float32)
                  + b2_ref[...])


def _k3(h1, W2t, b2row):
    grid = 100
    blk = N_EDGES // grid
    return pl.pallas_call(
        _k3_body,
        grid=(grid,),
        in_specs=[
            pl.BlockSpec((blk, D), lambda i: (i, 0)),
            pl.BlockSpec((D, D), lambda i: (0, 0)),
            pl.BlockSpec((1, D), lambda i: (0, 0)),
        ],
        out_specs=pl.BlockSpec((blk, D), lambda i: (i, 0)),
        out_shape=jax.ShapeDtypeStruct((N_EDGES, D), jnp.float32),
    )(h1, W2t, b2row)


@functools.partial(
    pl.kernel,
    out_type=jax.ShapeDtypeStruct((NPAD, D), jnp.float32),
    mesh=_mesh,
    compiler_params=_sc_params,
    scratch_types=[
        pltpu.VMEM((K4_CE,), jnp.int32),
        pltpu.VMEM((IDN,), jnp.int32),
        pltpu.VMEM((IDN,), jnp.int32),
        pltpu.VMEM((IDN, D), jnp.float32),
        pltpu.VMEM((NPW + 8, D), jnp.float32),
        pltpu.SemaphoreType.DMA,
    ],
)
def _k4(h2_hbm, dst_hbm, out_hbm, dstbuf, idbuf, ldbuf, rows, acc, sem):
    wid = lax.axis_index("s") * NC + lax.axis_index("c")
    lo = wid * NPW
    neg = jnp.full((L,), -jnp.inf, dtype=jnp.float32)
    trash_v = jnp.full((L,), TRASH, dtype=jnp.int32)
    iota = lax.iota(jnp.int32, L)
    ones = jnp.full((L,), 1, jnp.int32)
    zeros = jnp.zeros((L,), jnp.int32)
    npw_v = jnp.full((L,), NPW, jnp.int32)
    lov = jnp.full((L,), lo, jnp.int32)

    def initacc(r, c):
        for j in range(D // L):
            acc[r, pl.ds(j * L, L)] = neg
        return c

    lax.fori_loop(0, NPW + 8, initacc, 0)
    for k in range(IDN // L):
        idbuf[pl.ds(k * L, L)] = zeros
        ldbuf[pl.ds(k * L, L)] = trash_v

    def flush():
        pltpu.async_copy(h2_hbm.at[idbuf], rows, sem).wait()

        def maxgroup(g, c):
            base = g * L
            ldv = ldbuf[pl.ds(base, L)]
            for lane in range(L):
                ld = ldv[lane]
                for j in range(D // L):
                    sl = pl.ds(j * L, L)
                    acc[ld, sl] = jnp.maximum(acc[ld, sl],
                                              rows[base + lane, sl])
            return c

        lax.fori_loop(0, IDN // L, maxgroup, 0)

    def chunk(ci, off):
        pltpu.sync_copy(dst_hbm.at[pl.ds(ci * K4_CE, K4_CE)], dstbuf)

        def step(i, off2):
            offu = off2
            for u in range(UNR):
                vbase = (i * UNR + u) * L
                dv = dstbuf[pl.ds(vbase, L)]
                ld = dv - lov
                m = (ld >= zeros) & (ld < npw_v)
                mi = jnp.where(m, ones, zeros)
                cs = plsc.cumsum(mi)
                pos = cs + jnp.full((L,), offu - 1, jnp.int32)
                eid = jnp.full((L,), ci * K4_CE + vbase, jnp.int32) + iota
                plsc.store_scatter(idbuf, [pos], eid, mask=m)
                plsc.store_scatter(ldbuf, [pos], ld, mask=m)
                offu = offu + cs[L - 1]
            do_flush = offu >= GB

            @pl.when(do_flush)
            def _():
                flush()

            return jnp.where(do_flush, 0, offu)

        return lax.fori_loop(0, K4_CE // (UNR * L), step, off)

    off = lax.fori_loop(0, N_EDGES // K4_CE, chunk, jnp.int32(0))

    @pl.when(off > 0)
    def _():
        flush()

    pltpu.sync_copy(acc.at[pl.ds(0, NPW)], out_hbm.at[pl.ds(lo, NPW)])


def _k5_body(agg_ref, g_ref, b_ref, o_ref):
    a = agg_ref[...]
    rows = lax.broadcasted_iota(jnp.int32, (NPAD, 1), 0)
    valid = rows < N_NODES
    a = jnp.where(jnp.isfinite(a), a, 0.0)
    a = jnp.where(valid, a, 0.0)
    inv_n = 1.0 / N_NODES
    mean = jnp.sum(a, axis=0, keepdims=True) * inv_n
    d = jnp.where(valid, a - mean, 0.0)
    var = jnp.sum(d * d, axis=0, keepdims=True) * inv_n
    scale = g_ref[...] * lax.rsqrt(var + 1e-5)
    o_ref[...] = (a[0:N_NODES] - mean) * scale + b_ref[...]


def _k5(aggp, grow, brow):
    return pl.pallas_call(
        _k5_body,
        out_shape=jax.ShapeDtypeStruct((N_NODES, D), jnp.float32),
    )(aggp, grow, brow)


@jax.jit
def kernel(x, edge_index, edge_attr, W1, b1, W2, b2, gamma, beta):
    W1a = W1[:, :D]
    W1b = W1[:, D:]
    Wd = (W1a - W1b).T
    Wb = W1b.T
    src = edge_index[0]
    dst = edge_index[1]
    A, B = _k1(x, Wd, Wb, b1[None, :])
    h1 = _k2(A, B, dst, src)
    h2 = _k3(h1, W2.T, b2[None, :])
    aggp = _k4(h2, dst)
    out = _k5(aggp, gamma[None, :], beta[None, :])
    return (out, edge_index, edge_attr)
